# speedup vs baseline: 23.6476x; 23.6476x over previous
import jax
import jax.numpy as jnp
from jax.experimental import pallas as pl
from jax.experimental.pallas import tpu as pltpu

_MAX_LEN = 1024
_D = 128
_BI = 8


def _body(table_ref, out_ref):
    i0 = pl.program_id(0) * _BI
    for k in range(_BI):
        start = (_MAX_LEN - 1) - (i0 + k)
        out_ref[k] = table_ref[pl.ds(start, _MAX_LEN), :]


def kernel(x, table):
    del x
    s = _MAX_LEN
    return pl.pallas_call(
        _body,
        grid=(s // _BI,),
        in_specs=[
            pl.BlockSpec((2 * s - 1, _D), lambda i: (0, 0)),
        ],
        out_specs=pl.BlockSpec((_BI, s, _D), lambda i: (i, 0, 0)),
        out_shape=jax.ShapeDtypeStruct((s, s, _D), jnp.float32),
    )(table)
